# baseline (speedup 1.0000x reference)
import functools

import jax
import jax.numpy as jnp
from jax import lax
from jax.experimental import pallas as pl
from jax.experimental.pallas import tpu as pltpu
from jax.experimental.pallas import tpu_sc as plsc

N = 10000
E = 320000
CHUNK = 128
NCHUNKS = E // CHUNK
NCORES = 2
NSUB = 16
NW = NCORES * NSUB
CPT = NCHUNKS // NW
NEXTRA = NCHUNKS - CPT * NW
RPT = N // NSUB
BLK = 2000
GRID = N // BLK
DEGW = 8


def _make_agg(d_feat):
    mesh = plsc.VectorSubcoreMesh(core_axis_name="c", subcore_axis_name="s")
    nbuf = {64: 3, 32: 13, 16: 13}[d_feat]

    @functools.partial(
        pl.kernel,
        out_type=jax.ShapeDtypeStruct((N, 128), jnp.float32),
        mesh=mesh,
        compiler_params=pltpu.CompilerParams(use_tc_tiling_on_sc=False,
                                             needs_layout_passes=False),
        scratch_types=[
            pltpu.VMEM((CPT + 1, CHUNK), jnp.int32),
            pltpu.VMEM((CPT + 1, CHUNK), jnp.int32),
            pltpu.VMEM((nbuf, CHUNK, d_feat), jnp.float32),
            pltpu.VMEM_SHARED((N, d_feat), jnp.float32),
            pltpu.VMEM_SHARED((N, d_feat), jnp.float32),
            [pltpu.SemaphoreType.DMA] * nbuf,
        ],
    )
    def agg(src_hbm, dst_hbm, y_hbm, zeros_hbm, out_hbm,
            src_v, dst_v, rows_v, acc_sh, ytab_sh, gsems):
        c = lax.axis_index("c")
        s = lax.axis_index("s")
        wid = c * NSUB + s
        row0 = s * RPT
        pltpu.sync_copy(src_hbm.at[pl.ds(wid * CPT, CPT)],
                        src_v.at[pl.ds(0, CPT)])
        pltpu.sync_copy(dst_hbm.at[pl.ds(wid * CPT, CPT)],
                        dst_v.at[pl.ds(0, CPT)])

        @pl.when(wid < NEXTRA)
        def _():
            pltpu.sync_copy(src_hbm.at[pl.ds(NW * CPT + wid, 1)],
                            src_v.at[pl.ds(CPT, 1)])
            pltpu.sync_copy(dst_hbm.at[pl.ds(NW * CPT + wid, 1)],
                            dst_v.at[pl.ds(CPT, 1)])

        pltpu.sync_copy(y_hbm.at[pl.ds(row0, RPT), pl.ds(0, d_feat)],
                        ytab_sh.at[pl.ds(row0, RPT)])
        pltpu.sync_copy(zeros_hbm, acc_sh.at[pl.ds(row0, RPT)])
        plsc.subcore_barrier()

        def gather(i, b):
            pltpu.async_copy(ytab_sh.at[src_v.at[i]], rows_v.at[b], gsems[b])

        for b in range(nbuf):
            gather(b, b)

        @pl.loop(0, CPT, step=nbuf)
        def _(j):
            for b in range(nbuf):
                i = j + b
                pltpu.make_async_copy(ytab_sh.at[src_v.at[i]],
                                      rows_v.at[b], gsems[b]).wait()
                pltpu.sync_copy(rows_v.at[b], acc_sh.at[dst_v.at[i]], add=True)

                @pl.when(i + nbuf < CPT)
                def _():
                    gather(i + nbuf, b)

        @pl.when(wid < NEXTRA)
        def _():
            pltpu.async_copy(ytab_sh.at[src_v.at[CPT]], rows_v.at[0],
                             gsems[0]).wait()
            pltpu.sync_copy(rows_v.at[0], acc_sh.at[dst_v.at[CPT]], add=True)

        plsc.subcore_barrier()
        pltpu.sync_copy(acc_sh.at[pl.ds(row0, RPT)],
                        out_hbm.at[pl.ds(row0, RPT), pl.ds(c * d_feat, d_feat)])

    return agg


def _make_deg():
    mesh = plsc.VectorSubcoreMesh(core_axis_name="c", subcore_axis_name="s")

    @functools.partial(
        pl.kernel,
        out_type=jax.ShapeDtypeStruct((N, 128), jnp.float32),
        mesh=mesh,
        compiler_params=pltpu.CompilerParams(use_tc_tiling_on_sc=False,
                                             needs_layout_passes=False),
        scratch_types=[
            pltpu.VMEM((CPT + 1, CHUNK), jnp.int32),
            pltpu.VMEM((CHUNK, DEGW), jnp.float32),
            pltpu.VMEM_SHARED((N, DEGW), jnp.float32),
            pltpu.SemaphoreType.DMA,
        ],
    )
    def deg(dst_hbm, ones_hbm, zeros_hbm, out_hbm, dst_v, ones_v, acc_sh, sem):
        c = lax.axis_index("c")
        s = lax.axis_index("s")
        wid = c * NSUB + s
        row0 = s * RPT
        pltpu.sync_copy(dst_hbm.at[pl.ds(wid * CPT, CPT)],
                        dst_v.at[pl.ds(0, CPT)])

        @pl.when(wid < NEXTRA)
        def _():
            pltpu.sync_copy(dst_hbm.at[pl.ds(NW * CPT + wid, 1)],
                            dst_v.at[pl.ds(CPT, 1)])

        pltpu.sync_copy(ones_hbm, ones_v)
        pltpu.sync_copy(zeros_hbm, acc_sh.at[pl.ds(row0, RPT)])
        plsc.subcore_barrier()

        @pl.loop(0, CPT)
        def _(j):
            pltpu.async_copy(ones_v, acc_sh.at[dst_v.at[j]], sem, add=True)

        @pl.when(wid < NEXTRA)
        def _():
            pltpu.async_copy(ones_v, acc_sh.at[dst_v.at[CPT]], sem, add=True)

        @pl.loop(0, CPT)
        def _(j):
            pltpu.make_async_copy(ones_v, acc_sh.at[dst_v.at[j]], sem).wait()

        @pl.when(wid < NEXTRA)
        def _():
            pltpu.make_async_copy(ones_v, acc_sh.at[dst_v.at[CPT]], sem).wait()

        plsc.subcore_barrier()
        pltpu.sync_copy(acc_sh.at[pl.ds(row0, RPT)],
                        out_hbm.at[pl.ds(row0, RPT), pl.ds(c * 64, DEGW)])

    return deg


_deg = _make_deg()


def _stage_a(degp, x, W1p):
    def body(degp_ref, x_ref, w_ref, y_ref, dis_ref):
        deg = degp_ref[:, 0:1] + degp_ref[:, 64:65] + 1.0
        dis = lax.rsqrt(deg)
        dis_ref[...] = dis
        y_ref[...] = dis * jnp.dot(x_ref[...], w_ref[...],
                                   preferred_element_type=jnp.float32)

    return pl.pallas_call(
        body,
        grid=(GRID,),
        in_specs=[
            pl.BlockSpec((BLK, 128), lambda i: (i, 0)),
            pl.BlockSpec((BLK, 128), lambda i: (i, 0)),
            pl.BlockSpec((128, 128), lambda i: (0, 0)),
        ],
        out_specs=[
            pl.BlockSpec((BLK, 128), lambda i: (i, 0)),
            pl.BlockSpec((BLK, 1), lambda i: (i, 0)),
        ],
        out_shape=[
            jax.ShapeDtypeStruct((N, 128), jnp.float32),
            jax.ShapeDtypeStruct((N, 1), jnp.float32),
        ],
    )(degp, x, W1p)


def _stage_b(q, y, dis, b, Wn, d_in):
    d_out = Wn.shape[1]

    def body(q_ref, y_ref, dis_ref, b_ref, w_ref, f_ref, yn_ref):
        agg = q_ref[:, 0:d_in] + q_ref[:, d_in:2 * d_in] + y_ref[:, 0:d_in]
        f = jnp.maximum(dis_ref[...] * agg + b_ref[...], 0.0)
        f_ref[...] = f
        yn_ref[...] = dis_ref[...] * jnp.dot(f, w_ref[...],
                                             preferred_element_type=jnp.float32)

    return pl.pallas_call(
        body,
        grid=(GRID,),
        in_specs=[
            pl.BlockSpec((BLK, 128), lambda i: (i, 0)),
            pl.BlockSpec((BLK, 128), lambda i: (i, 0)),
            pl.BlockSpec((BLK, 1), lambda i: (i, 0)),
            pl.BlockSpec((1, d_in), lambda i: (0, 0)),
            pl.BlockSpec((d_in, 128), lambda i: (0, 0)),
        ],
        out_specs=[
            pl.BlockSpec((BLK, d_in), lambda i: (i, 0)),
            pl.BlockSpec((BLK, 128), lambda i: (i, 0)),
        ],
        out_shape=[
            jax.ShapeDtypeStruct((N, d_in), jnp.float32),
            jax.ShapeDtypeStruct((N, 128), jnp.float32),
        ],
    )(q, y, dis, b, Wn)


def _stage_c(q3, y3, dis, b3, f1, f2, Wa, Wb, Wc, bfc):
    def body(q_ref, y_ref, dis_ref, b_ref, f1_ref, f2_ref,
             wa_ref, wb_ref, wc_ref, bfc_ref, out_ref):
        agg = q_ref[:, 0:16] + q_ref[:, 16:32] + y_ref[:, 0:16]
        f3 = jnp.maximum(dis_ref[...] * agg + b_ref[...], 0.0)
        acc = jnp.dot(f1_ref[...], wa_ref[...], preferred_element_type=jnp.float32)
        acc = acc + jnp.dot(f2_ref[...], wb_ref[...], preferred_element_type=jnp.float32)
        acc = acc + jnp.dot(f3, wc_ref[...], preferred_element_type=jnp.float32)
        out_ref[...] = jnp.maximum(acc + bfc_ref[...], 0.0)

    return pl.pallas_call(
        body,
        grid=(GRID,),
        in_specs=[
            pl.BlockSpec((BLK, 128), lambda i: (i, 0)),
            pl.BlockSpec((BLK, 128), lambda i: (i, 0)),
            pl.BlockSpec((BLK, 1), lambda i: (i, 0)),
            pl.BlockSpec((1, 16), lambda i: (0, 0)),
            pl.BlockSpec((BLK, 64), lambda i: (i, 0)),
            pl.BlockSpec((BLK, 32), lambda i: (i, 0)),
            pl.BlockSpec((64, 16), lambda i: (0, 0)),
            pl.BlockSpec((32, 16), lambda i: (0, 0)),
            pl.BlockSpec((16, 16), lambda i: (0, 0)),
            pl.BlockSpec((1, 16), lambda i: (0, 0)),
        ],
        out_specs=pl.BlockSpec((BLK, 16), lambda i: (i, 0)),
        out_shape=jax.ShapeDtypeStruct((N, 16), jnp.float32),
    )(q3, y3, dis, b3, f1, f2, Wa, Wb, Wc, bfc)


_agg = {d: _make_agg(d) for d in (16, 32, 64)}


def kernel(edges, features, W1, b1, W2, b2, W3, b3, Wfc, bfc):
    srcp = edges[0].astype(jnp.int32).reshape(NCHUNKS, CHUNK)
    dstp = edges[1].astype(jnp.int32).reshape(NCHUNKS, CHUNK)

    ones_c = jnp.ones((CHUNK, DEGW), jnp.float32)
    zdeg = jnp.zeros((RPT, DEGW), jnp.float32)
    z16 = jnp.zeros((RPT, 16), jnp.float32)
    z32 = jnp.zeros((RPT, 32), jnp.float32)
    z64 = jnp.zeros((RPT, 64), jnp.float32)
    W1p = jnp.pad(W1, ((0, 0), (0, 64)))
    W2p = jnp.pad(W2, ((0, 0), (0, 96)))
    W3p = jnp.pad(W3, ((0, 0), (0, 112)))

    degp = _deg(dstp, ones_c, zdeg)
    y1, dis = _stage_a(degp, features, W1p)
    q1 = _agg[64](srcp, dstp, y1, z64)
    f1, y2 = _stage_b(q1, y1, dis, b1.reshape(1, -1), W2p, 64)
    q2 = _agg[32](srcp, dstp, y2, z32)
    f2, y3 = _stage_b(q2, y2, dis, b2.reshape(1, -1), W3p, 32)
    q3 = _agg[16](srcp, dstp, y3, z16)
    ret = _stage_c(q3, y3, dis, b3.reshape(1, -1), f1, f2,
                   Wfc[:64], Wfc[64:96], Wfc[96:112], bfc.reshape(1, -1))
    return ret


# speedup vs baseline: 1.0015x; 1.0015x over previous
import functools

import jax
import jax.numpy as jnp
from jax import lax
from jax.experimental import pallas as pl
from jax.experimental.pallas import tpu as pltpu
from jax.experimental.pallas import tpu_sc as plsc

N = 10000
E = 320000
CHUNK = 128
NCHUNKS = E // CHUNK
NCORES = 2
NSUB = 16
NW = NCORES * NSUB
CPT = NCHUNKS // NW
NEXTRA = NCHUNKS - CPT * NW
RPT = N // NSUB
BLK = 2000
GRID = N // BLK
DEGW = 8


def _make_agg(d_feat):
    mesh = plsc.VectorSubcoreMesh(core_axis_name="c", subcore_axis_name="s")
    nbuf = {64: 3, 32: 6, 16: 6}[d_feat]

    @functools.partial(
        pl.kernel,
        out_type=jax.ShapeDtypeStruct((N, 128), jnp.float32),
        mesh=mesh,
        compiler_params=pltpu.CompilerParams(use_tc_tiling_on_sc=False,
                                             needs_layout_passes=False),
        scratch_types=[
            pltpu.VMEM((CPT + 1, CHUNK), jnp.int32),
            pltpu.VMEM((CPT + 1, CHUNK), jnp.int32),
            pltpu.VMEM((nbuf, CHUNK, d_feat), jnp.float32),
            pltpu.VMEM_SHARED((N, d_feat), jnp.float32),
            pltpu.VMEM_SHARED((N, d_feat), jnp.float32),
            [pltpu.SemaphoreType.DMA] * nbuf,
        ],
    )
    def agg(src_hbm, dst_hbm, y_hbm, zeros_hbm, out_hbm,
            src_v, dst_v, rows_v, acc_sh, ytab_sh, gsems):
        c = lax.axis_index("c")
        s = lax.axis_index("s")
        wid = c * NSUB + s
        row0 = s * RPT
        pltpu.sync_copy(src_hbm.at[pl.ds(wid * CPT, CPT)],
                        src_v.at[pl.ds(0, CPT)])
        pltpu.sync_copy(dst_hbm.at[pl.ds(wid * CPT, CPT)],
                        dst_v.at[pl.ds(0, CPT)])

        @pl.when(wid < NEXTRA)
        def _():
            pltpu.sync_copy(src_hbm.at[pl.ds(NW * CPT + wid, 1)],
                            src_v.at[pl.ds(CPT, 1)])
            pltpu.sync_copy(dst_hbm.at[pl.ds(NW * CPT + wid, 1)],
                            dst_v.at[pl.ds(CPT, 1)])

        pltpu.sync_copy(y_hbm.at[pl.ds(row0, RPT), pl.ds(0, d_feat)],
                        ytab_sh.at[pl.ds(row0, RPT)])
        pltpu.sync_copy(zeros_hbm, acc_sh.at[pl.ds(row0, RPT)])
        plsc.subcore_barrier()

        def gather(i, b):
            pltpu.async_copy(ytab_sh.at[src_v.at[i]], rows_v.at[b], gsems[b])

        for b in range(nbuf):
            gather(b, b)

        @pl.loop(0, CPT, step=nbuf)
        def _(j):
            for b in range(nbuf):
                i = j + b
                pltpu.make_async_copy(ytab_sh.at[src_v.at[i]],
                                      rows_v.at[b], gsems[b]).wait()
                pltpu.sync_copy(rows_v.at[b], acc_sh.at[dst_v.at[i]], add=True)

                @pl.when(i + nbuf < CPT)
                def _():
                    gather(i + nbuf, b)

        @pl.when(wid < NEXTRA)
        def _():
            pltpu.async_copy(ytab_sh.at[src_v.at[CPT]], rows_v.at[0],
                             gsems[0]).wait()
            pltpu.sync_copy(rows_v.at[0], acc_sh.at[dst_v.at[CPT]], add=True)

        plsc.subcore_barrier()
        pltpu.sync_copy(acc_sh.at[pl.ds(row0, RPT)],
                        out_hbm.at[pl.ds(row0, RPT), pl.ds(c * d_feat, d_feat)])

    return agg


def _make_deg():
    mesh = plsc.VectorSubcoreMesh(core_axis_name="c", subcore_axis_name="s")

    @functools.partial(
        pl.kernel,
        out_type=jax.ShapeDtypeStruct((N, 128), jnp.float32),
        mesh=mesh,
        compiler_params=pltpu.CompilerParams(use_tc_tiling_on_sc=False,
                                             needs_layout_passes=False),
        scratch_types=[
            pltpu.VMEM((CPT + 1, CHUNK), jnp.int32),
            pltpu.VMEM((CHUNK, DEGW), jnp.float32),
            pltpu.VMEM_SHARED((N, DEGW), jnp.float32),
            pltpu.SemaphoreType.DMA,
        ],
    )
    def deg(dst_hbm, ones_hbm, zeros_hbm, out_hbm, dst_v, ones_v, acc_sh, sem):
        c = lax.axis_index("c")
        s = lax.axis_index("s")
        wid = c * NSUB + s
        row0 = s * RPT
        pltpu.sync_copy(dst_hbm.at[pl.ds(wid * CPT, CPT)],
                        dst_v.at[pl.ds(0, CPT)])

        @pl.when(wid < NEXTRA)
        def _():
            pltpu.sync_copy(dst_hbm.at[pl.ds(NW * CPT + wid, 1)],
                            dst_v.at[pl.ds(CPT, 1)])

        pltpu.sync_copy(ones_hbm, ones_v)
        pltpu.sync_copy(zeros_hbm, acc_sh.at[pl.ds(row0, RPT)])
        plsc.subcore_barrier()

        @pl.loop(0, CPT)
        def _(j):
            pltpu.async_copy(ones_v, acc_sh.at[dst_v.at[j]], sem, add=True)

        @pl.when(wid < NEXTRA)
        def _():
            pltpu.async_copy(ones_v, acc_sh.at[dst_v.at[CPT]], sem, add=True)

        @pl.loop(0, CPT)
        def _(j):
            pltpu.make_async_copy(ones_v, acc_sh.at[dst_v.at[j]], sem).wait()

        @pl.when(wid < NEXTRA)
        def _():
            pltpu.make_async_copy(ones_v, acc_sh.at[dst_v.at[CPT]], sem).wait()

        plsc.subcore_barrier()
        pltpu.sync_copy(acc_sh.at[pl.ds(row0, RPT)],
                        out_hbm.at[pl.ds(row0, RPT), pl.ds(c * 64, DEGW)])

    return deg


_deg = _make_deg()


def _stage_a(degp, x, W1p):
    def body(degp_ref, x_ref, w_ref, y_ref, dis_ref):
        deg = degp_ref[:, 0:1] + degp_ref[:, 64:65] + 1.0
        dis = lax.rsqrt(deg)
        dis_ref[...] = dis
        y_ref[...] = dis * jnp.dot(x_ref[...], w_ref[...],
                                   preferred_element_type=jnp.float32)

    return pl.pallas_call(
        body,
        grid=(GRID,),
        in_specs=[
            pl.BlockSpec((BLK, 128), lambda i: (i, 0)),
            pl.BlockSpec((BLK, 128), lambda i: (i, 0)),
            pl.BlockSpec((128, 128), lambda i: (0, 0)),
        ],
        out_specs=[
            pl.BlockSpec((BLK, 128), lambda i: (i, 0)),
            pl.BlockSpec((BLK, 1), lambda i: (i, 0)),
        ],
        out_shape=[
            jax.ShapeDtypeStruct((N, 128), jnp.float32),
            jax.ShapeDtypeStruct((N, 1), jnp.float32),
        ],
    )(degp, x, W1p)


def _stage_b(q, y, dis, b, Wn, d_in):
    d_out = Wn.shape[1]

    def body(q_ref, y_ref, dis_ref, b_ref, w_ref, f_ref, yn_ref):
        agg = q_ref[:, 0:d_in] + q_ref[:, d_in:2 * d_in] + y_ref[:, 0:d_in]
        f = jnp.maximum(dis_ref[...] * agg + b_ref[...], 0.0)
        f_ref[...] = f
        yn_ref[...] = dis_ref[...] * jnp.dot(f, w_ref[...],
                                             preferred_element_type=jnp.float32)

    return pl.pallas_call(
        body,
        grid=(GRID,),
        in_specs=[
            pl.BlockSpec((BLK, 128), lambda i: (i, 0)),
            pl.BlockSpec((BLK, 128), lambda i: (i, 0)),
            pl.BlockSpec((BLK, 1), lambda i: (i, 0)),
            pl.BlockSpec((1, d_in), lambda i: (0, 0)),
            pl.BlockSpec((d_in, 128), lambda i: (0, 0)),
        ],
        out_specs=[
            pl.BlockSpec((BLK, d_in), lambda i: (i, 0)),
            pl.BlockSpec((BLK, 128), lambda i: (i, 0)),
        ],
        out_shape=[
            jax.ShapeDtypeStruct((N, d_in), jnp.float32),
            jax.ShapeDtypeStruct((N, 128), jnp.float32),
        ],
    )(q, y, dis, b, Wn)


def _stage_c(q3, y3, dis, b3, f1, f2, Wa, Wb, Wc, bfc):
    def body(q_ref, y_ref, dis_ref, b_ref, f1_ref, f2_ref,
             wa_ref, wb_ref, wc_ref, bfc_ref, out_ref):
        agg = q_ref[:, 0:16] + q_ref[:, 16:32] + y_ref[:, 0:16]
        f3 = jnp.maximum(dis_ref[...] * agg + b_ref[...], 0.0)
        acc = jnp.dot(f1_ref[...], wa_ref[...], preferred_element_type=jnp.float32)
        acc = acc + jnp.dot(f2_ref[...], wb_ref[...], preferred_element_type=jnp.float32)
        acc = acc + jnp.dot(f3, wc_ref[...], preferred_element_type=jnp.float32)
        out_ref[...] = jnp.maximum(acc + bfc_ref[...], 0.0)

    return pl.pallas_call(
        body,
        grid=(GRID,),
        in_specs=[
            pl.BlockSpec((BLK, 128), lambda i: (i, 0)),
            pl.BlockSpec((BLK, 128), lambda i: (i, 0)),
            pl.BlockSpec((BLK, 1), lambda i: (i, 0)),
            pl.BlockSpec((1, 16), lambda i: (0, 0)),
            pl.BlockSpec((BLK, 64), lambda i: (i, 0)),
            pl.BlockSpec((BLK, 32), lambda i: (i, 0)),
            pl.BlockSpec((64, 16), lambda i: (0, 0)),
            pl.BlockSpec((32, 16), lambda i: (0, 0)),
            pl.BlockSpec((16, 16), lambda i: (0, 0)),
            pl.BlockSpec((1, 16), lambda i: (0, 0)),
        ],
        out_specs=pl.BlockSpec((BLK, 16), lambda i: (i, 0)),
        out_shape=jax.ShapeDtypeStruct((N, 16), jnp.float32),
    )(q3, y3, dis, b3, f1, f2, Wa, Wb, Wc, bfc)


_agg = {d: _make_agg(d) for d in (16, 32, 64)}


def kernel(edges, features, W1, b1, W2, b2, W3, b3, Wfc, bfc):
    srcp = edges[0].astype(jnp.int32).reshape(NCHUNKS, CHUNK)
    dstp = edges[1].astype(jnp.int32).reshape(NCHUNKS, CHUNK)

    ones_c = jnp.ones((CHUNK, DEGW), jnp.float32)
    zdeg = jnp.zeros((RPT, DEGW), jnp.float32)
    z16 = jnp.zeros((RPT, 16), jnp.float32)
    z32 = jnp.zeros((RPT, 32), jnp.float32)
    z64 = jnp.zeros((RPT, 64), jnp.float32)
    W1p = jnp.pad(W1, ((0, 0), (0, 64)))
    W2p = jnp.pad(W2, ((0, 0), (0, 96)))
    W3p = jnp.pad(W3, ((0, 0), (0, 112)))

    degp = _deg(dstp, ones_c, zdeg)
    y1, dis = _stage_a(degp, features, W1p)
    q1 = _agg[64](srcp, dstp, y1, z64)
    f1, y2 = _stage_b(q1, y1, dis, b1.reshape(1, -1), W2p, 64)
    q2 = _agg[32](srcp, dstp, y2, z32)
    f2, y3 = _stage_b(q2, y2, dis, b2.reshape(1, -1), W3p, 32)
    q3 = _agg[16](srcp, dstp, y3, z16)
    ret = _stage_c(q3, y3, dis, b3.reshape(1, -1), f1, f2,
                   Wfc[:64], Wfc[64:96], Wfc[96:112], bfc.reshape(1, -1))
    return ret
